# baseline (speedup 1.0000x reference)
import itertools
from functools import partial

import numpy as np
import jax
import jax.numpy as jnp
from jax.experimental import pallas as pl

_NX, _NY = 6, 6
_NB = _NX * _NY


def _adj_np():
    adj = []
    for i, j in itertools.product(range(_NX), range(_NY)):
        for dx, dy in [(-1, 0), (0, -1), (0, 1), (1, 0)]:
            x, y = i + dx, j + dy
            if 0 <= x < _NX and 0 <= y < _NY:
                adj.append((j * _NX + i, y * _NX + x))
    return np.array(adj, dtype=np.int32)


_ADJ = _adj_np()
_NE = _ADJ.shape[0]

_SCAT = np.zeros((_NE, _NB * _NB), dtype=np.float32)
_SCAT[np.arange(_NE), _ADJ[:, 0] * _NB + _ADJ[:, 1]] = 1.0

_PERM = np.full((5 * 8 * 64,), -1, dtype=np.int32)
for _i in range(5):
    for _j in range(5):
        for _c in range(64):
            _PERM[(_i * 8 + _j) * 64 + _c] = _c * 25 + _i * 5 + _j


def _trunk_body(xp_ref, tm_ref, b12_ref, w3_ref, b3_ref, w4_ref, b4_ref,
                out_ref):
    x = xp_ref[0]
    xs = jnp.concatenate([x[:, dx:dx + 128] for dx in range(9)], axis=0)
    y = jnp.dot(tm_ref[...], xs, preferred_element_type=jnp.float32)
    p = jnp.max(y.reshape(32, 30, 4, 128), axis=2)
    p = jnp.max(p.reshape(32, 30, 32, 4), axis=3)
    pc = p.reshape(32, 960)
    ii = jax.lax.broadcasted_iota(jnp.int32, (32, 32), 0)
    jj = jax.lax.broadcasted_iota(jnp.int32, (32, 32), 1)
    eye = jnp.where(ii == jj, 1.0, 0.0).astype(jnp.float32)
    x2 = jax.lax.dot_general(pc, eye, (((0,), (0,)), ((), ())),
                             preferred_element_type=jnp.float32)
    x2 = x2 + b12_ref[0][None, :]
    x2 = jnp.concatenate([x2, jnp.zeros((8, 32), jnp.float32)], axis=0)
    a3 = jnp.zeros((832, 64), jnp.float32) + b3_ref[0][None, :]
    for t in range(25):
        off = (t // 5) * 32 + t % 5
        a3 = a3 + jnp.dot(x2[off:off + 832, :], w3_ref[t],
                          preferred_element_type=jnp.float32)
    a3 = jnp.concatenate([a3, jnp.zeros((8, 64), jnp.float32)], axis=0)
    a4 = jnp.zeros((704, 64), jnp.float32) + b4_ref[0][None, :]
    for t in range(25):
        off = (t // 5) * 32 + t % 5
        a4 = a4 + jnp.dot(a3[off:off + 704, :], w4_ref[t],
                          preferred_element_type=jnp.float32)
    q = jnp.max(a4.reshape(22, 8, 4, 64), axis=2)
    q = q[0:20]
    q = jnp.max(q.reshape(5, 4, 8, 64), axis=1)
    out_ref[0] = q


def _heads_body(emb_ref, wc1_ref, bc1_ref, wc2_ref, bc2_ref,
                we1_ref, be1_ref, we2_ref, be2_ref, s_ref,
                a_ref, ie_ref):
    emb = emb_ref[...]
    hc = jnp.maximum(
        jnp.dot(emb, wc1_ref[...], preferred_element_type=jnp.float32)
        + bc1_ref[0][None, :], 0.0)
    ic = jax.nn.sigmoid(
        jnp.dot(hc, wc2_ref[...], preferred_element_type=jnp.float32)
        + bc2_ref[0][None, :])
    a_ref[...] = jnp.dot(ic, s_ref[...], preferred_element_type=jnp.float32)
    he = jnp.maximum(
        jnp.dot(emb, we1_ref[...], preferred_element_type=jnp.float32)
        + be1_ref[0][None, :], 0.0)
    ie_ref[...] = jax.nn.sigmoid(
        jnp.dot(he, we2_ref[...], preferred_element_type=jnp.float32)
        + be2_ref[0][None, :])


def _graph_body(a_ref, ier_ref, iec_ref, o_ref):
    r = a_ref[0]
    for _ in range(6):
        acc = r
        for k in range(_NB):
            acc = jnp.maximum(acc, r[:, k:k + 1] * r[k:k + 1, :])
        r = acc
    pair = r * ier_ref[0] * iec_ref[0]
    ii = jax.lax.broadcasted_iota(jnp.int32, (_NB, _NB), 0)
    jj = jax.lax.broadcasted_iota(jnp.int32, (_NB, _NB), 1)
    pair = jnp.where(ii != jj, pair, 0.0)
    o_ref[0] = jnp.max(pair, axis=(0, 1), keepdims=True)


def kernel(input, W1, b1, W2, b2, W3, b3, W4, b4, We1, be1, We2, be2,
           Wc1, bc1, Wc2, bc2):
    B = input.shape[0]
    f32 = jnp.float32

    w12 = jnp.zeros((32, 9, 9), f32)
    for u1 in range(5):
        for v1 in range(5):
            w12 = w12.at[:, u1:u1 + 5, v1:v1 + 5].add(
                jnp.einsum('ocuv,c->ouv', W2, W1[:, 0, u1, v1]))
    tm = jnp.zeros((32, 120, 9, 128), f32)
    ivec = jnp.arange(120)
    for dy in range(9):
        tm = tm.at[:, ivec, :, ivec + dy].set(
            jnp.broadcast_to(w12[:, dy, :][None], (120, 32, 9)))
    tm = tm.reshape(3840, 1152)
    b12 = (b2 + W2.sum(axis=(2, 3)) @ b1).reshape(1, 32)
    w3m = W3.transpose(2, 3, 1, 0).reshape(25, 32, 64)
    w4m = W4.transpose(2, 3, 1, 0).reshape(25, 64, 64)

    perm = jnp.asarray(_PERM)
    wc1h = jnp.where(perm[:, None] >= 0, Wc1[jnp.maximum(perm, 0)], 0.0)
    we1h = jnp.where(perm[:, None] >= 0, We1[jnp.maximum(perm, 0)], 0.0)

    xp = jnp.pad(input[:, 0], ((0, 0), (0, 0), (0, 8)))

    emb3 = pl.pallas_call(
        _trunk_body,
        grid=(B,),
        in_specs=[
            pl.BlockSpec((1, 128, 136), lambda b: (b, 0, 0)),
            pl.BlockSpec((3840, 1152), lambda b: (0, 0)),
            pl.BlockSpec((1, 32), lambda b: (0, 0)),
            pl.BlockSpec((25, 32, 64), lambda b: (0, 0, 0)),
            pl.BlockSpec((1, 64), lambda b: (0, 0)),
            pl.BlockSpec((25, 64, 64), lambda b: (0, 0, 0)),
            pl.BlockSpec((1, 64), lambda b: (0, 0)),
        ],
        out_specs=pl.BlockSpec((1, 5, 8, 64), lambda b: (b, 0, 0, 0)),
        out_shape=jax.ShapeDtypeStruct((B, 5, 8, 64), f32),
    )(xp, tm, b12, w3m, b3.reshape(1, 64), w4m, b4.reshape(1, 64))

    emb = emb3.reshape(B, 2560)

    a_flat, ie = pl.pallas_call(
        _heads_body,
        out_shape=(jax.ShapeDtypeStruct((B, _NB * _NB), f32),
                   jax.ShapeDtypeStruct((B, _NB), f32)),
    )(emb, wc1h, bc1.reshape(1, 256), Wc2, bc2.reshape(1, _NE),
      we1h, be1.reshape(1, 256), We2, be2.reshape(1, _NB),
      jnp.asarray(_SCAT))

    a3 = a_flat.reshape(B, _NB, _NB)
    ier = ie.reshape(B, 1, _NB)
    iec = ie.reshape(B, _NB, 1)
    out = pl.pallas_call(
        _graph_body,
        grid=(B,),
        in_specs=[
            pl.BlockSpec((1, _NB, _NB), lambda b: (b, 0, 0)),
            pl.BlockSpec((1, 1, _NB), lambda b: (b, 0, 0)),
            pl.BlockSpec((1, _NB, 1), lambda b: (b, 0, 0)),
        ],
        out_specs=pl.BlockSpec((1, 1, 1), lambda b: (b, 0, 0)),
        out_shape=jax.ShapeDtypeStruct((B, 1, 1), f32),
    )(a3, ier, iec)
    return out.reshape(B)


# speedup vs baseline: 1.5857x; 1.5857x over previous
import itertools
from functools import partial

import numpy as np
import jax
import jax.numpy as jnp
from jax.experimental import pallas as pl

_NX, _NY = 6, 6
_NB = _NX * _NY


def _adj_np():
    adj = []
    for i, j in itertools.product(range(_NX), range(_NY)):
        for dx, dy in [(-1, 0), (0, -1), (0, 1), (1, 0)]:
            x, y = i + dx, j + dy
            if 0 <= x < _NX and 0 <= y < _NY:
                adj.append((j * _NX + i, y * _NX + x))
    return np.array(adj, dtype=np.int32)


_ADJ = _adj_np()
_NE = _ADJ.shape[0]

_SCAT = np.zeros((_NE, _NB * _NB), dtype=np.float32)
_SCAT[np.arange(_NE), _ADJ[:, 0] * _NB + _ADJ[:, 1]] = 1.0

_PERM = np.full((5 * 8 * 64,), -1, dtype=np.int32)
for _i in range(5):
    for _j in range(5):
        for _c in range(64):
            _PERM[(_i * 8 + _j) * 64 + _c] = _c * 25 + _i * 5 + _j


def _trunk_body(xp_ref, tm_ref, b12_ref, w3_ref, b3_ref, w4_ref, b4_ref,
                out_ref):
    x = xp_ref[0]
    ps = []
    for t in range(15):
        xt = jnp.concatenate(
            [x[8 * t:8 * t + 16, dx:dx + 128] for dx in range(9)], axis=0)
        yt = jnp.dot(tm_ref[...], xt, preferred_element_type=jnp.float32)
        q = jnp.max(yt.reshape(32, 8, 32, 4), axis=3)
        q = jnp.max(q.reshape(32, 2, 4, 32), axis=2)
        ps.append(q)
    pc = jnp.concatenate(ps, axis=1).reshape(32, 960)
    ii = jax.lax.broadcasted_iota(jnp.int32, (32, 32), 0)
    jj = jax.lax.broadcasted_iota(jnp.int32, (32, 32), 1)
    eye = jnp.where(ii == jj, 1.0, 0.0).astype(jnp.float32)
    x2 = jax.lax.dot_general(pc, eye, (((0,), (0,)), ((), ())),
                             preferred_element_type=jnp.float32)
    x2 = x2 + b12_ref[0][None, :]
    x2 = jnp.concatenate([x2, jnp.zeros((8, 32), jnp.float32)], axis=0)
    a3 = jnp.zeros((832, 64), jnp.float32) + b3_ref[0][None, :]
    for dy in range(5):
        xc = jnp.concatenate(
            [x2[dy * 32 + dx:dy * 32 + dx + 832, :] for dx in range(5)],
            axis=1)
        a3 = a3 + jnp.dot(xc, w3_ref[dy * 160:(dy + 1) * 160, :],
                          preferred_element_type=jnp.float32)
    a3 = jnp.concatenate([a3, jnp.zeros((8, 64), jnp.float32)], axis=0)
    a4 = jnp.zeros((704, 64), jnp.float32) + b4_ref[0][None, :]
    for dy in range(5):
        xc = jnp.concatenate(
            [a3[dy * 32 + dx:dy * 32 + dx + 704, :] for dx in range(5)],
            axis=1)
        a4 = a4 + jnp.dot(xc, w4_ref[dy * 320:(dy + 1) * 320, :],
                          preferred_element_type=jnp.float32)
    q = jnp.max(a4.reshape(22, 8, 4, 64), axis=2)
    q = q[0:20]
    q = jnp.max(q.reshape(5, 4, 8, 64), axis=1)
    out_ref[0] = q


def _heads_body(emb_ref, wc1_ref, bc1_ref, wc2_ref, bc2_ref,
                we1_ref, be1_ref, we2_ref, be2_ref, s_ref,
                a_ref, ie_ref):
    emb = emb_ref[...]
    hc = jnp.maximum(
        jnp.dot(emb, wc1_ref[...], preferred_element_type=jnp.float32)
        + bc1_ref[0][None, :], 0.0)
    ic = jax.nn.sigmoid(
        jnp.dot(hc, wc2_ref[...], preferred_element_type=jnp.float32)
        + bc2_ref[0][None, :])
    a_ref[...] = jnp.dot(ic, s_ref[...], preferred_element_type=jnp.float32)
    he = jnp.maximum(
        jnp.dot(emb, we1_ref[...], preferred_element_type=jnp.float32)
        + be1_ref[0][None, :], 0.0)
    ie_ref[...] = jax.nn.sigmoid(
        jnp.dot(he, we2_ref[...], preferred_element_type=jnp.float32)
        + be2_ref[0][None, :])


def _graph_body(a_ref, ier_ref, iec_ref, o_ref):
    r = a_ref[0]
    for _ in range(6):
        acc = r
        for k in range(_NB):
            acc = jnp.maximum(acc, r[:, k:k + 1] * r[k:k + 1, :])
        r = acc
    pair = r * ier_ref[0] * iec_ref[0]
    ii = jax.lax.broadcasted_iota(jnp.int32, (_NB, _NB), 0)
    jj = jax.lax.broadcasted_iota(jnp.int32, (_NB, _NB), 1)
    pair = jnp.where(ii != jj, pair, 0.0)
    o_ref[0] = jnp.max(pair, axis=(0, 1), keepdims=True)


def kernel(input, W1, b1, W2, b2, W3, b3, W4, b4, We1, be1, We2, be2,
           Wc1, bc1, Wc2, bc2):
    B = input.shape[0]
    f32 = jnp.float32

    w12 = jnp.zeros((32, 9, 9), f32)
    for u1 in range(5):
        for v1 in range(5):
            w12 = w12.at[:, u1:u1 + 5, v1:v1 + 5].add(
                jnp.einsum('ocuv,c->ouv', W2, W1[:, 0, u1, v1]))
    z = jnp.concatenate([w12.transpose(0, 2, 1),
                         jnp.zeros((32, 9, 8), f32)], axis=2)
    tm = jnp.tile(z, (1, 1, 8))[:, :, :128].reshape(32, 9, 8, 16)
    tm = tm.transpose(0, 2, 1, 3).reshape(256, 144)
    b12 = (b2 + W2.sum(axis=(2, 3)) @ b1).reshape(1, 32)
    w3m = W3.transpose(2, 3, 1, 0).reshape(800, 64)
    w4m = W4.transpose(2, 3, 1, 0).reshape(1600, 64)

    def _hwc(w):
        w = w.reshape(64, 5, 5, 256).transpose(1, 2, 0, 3)
        w = jnp.pad(w, ((0, 0), (0, 3), (0, 0), (0, 0)))
        return w.reshape(2560, 256)

    wc1h = _hwc(Wc1)
    we1h = _hwc(We1)

    xp = jnp.pad(input[:, 0], ((0, 0), (0, 0), (0, 8)))

    emb3 = pl.pallas_call(
        _trunk_body,
        grid=(B,),
        in_specs=[
            pl.BlockSpec((1, 128, 136), lambda b: (b, 0, 0)),
            pl.BlockSpec((256, 144), lambda b: (0, 0)),
            pl.BlockSpec((1, 32), lambda b: (0, 0)),
            pl.BlockSpec((800, 64), lambda b: (0, 0)),
            pl.BlockSpec((1, 64), lambda b: (0, 0)),
            pl.BlockSpec((1600, 64), lambda b: (0, 0)),
            pl.BlockSpec((1, 64), lambda b: (0, 0)),
        ],
        out_specs=pl.BlockSpec((1, 5, 8, 64), lambda b: (b, 0, 0, 0)),
        out_shape=jax.ShapeDtypeStruct((B, 5, 8, 64), f32),
    )(xp, tm, b12, w3m, b3.reshape(1, 64), w4m, b4.reshape(1, 64))

    emb = emb3.reshape(B, 2560)

    a_flat, ie = pl.pallas_call(
        _heads_body,
        out_shape=(jax.ShapeDtypeStruct((B, _NB * _NB), f32),
                   jax.ShapeDtypeStruct((B, _NB), f32)),
    )(emb, wc1h, bc1.reshape(1, 256), Wc2, bc2.reshape(1, _NE),
      we1h, be1.reshape(1, 256), We2, be2.reshape(1, _NB),
      jnp.asarray(_SCAT))

    a3 = a_flat.reshape(B, _NB, _NB)
    ier = ie.reshape(B, 1, _NB)
    iec = ie.reshape(B, _NB, 1)
    out = pl.pallas_call(
        _graph_body,
        grid=(B,),
        in_specs=[
            pl.BlockSpec((1, _NB, _NB), lambda b: (b, 0, 0)),
            pl.BlockSpec((1, 1, _NB), lambda b: (b, 0, 0)),
            pl.BlockSpec((1, _NB, 1), lambda b: (b, 0, 0)),
        ],
        out_specs=pl.BlockSpec((1, 1, 1), lambda b: (b, 0, 0)),
        out_shape=jax.ShapeDtypeStruct((B, 1, 1), f32),
    )(a3, ier, iec)
    return out.reshape(B)


# speedup vs baseline: 1.6985x; 1.0712x over previous
import itertools
from functools import partial

import numpy as np
import jax
import jax.numpy as jnp
from jax.experimental import pallas as pl

_NX, _NY = 6, 6
_NB = _NX * _NY


def _adj_np():
    adj = []
    for i, j in itertools.product(range(_NX), range(_NY)):
        for dx, dy in [(-1, 0), (0, -1), (0, 1), (1, 0)]:
            x, y = i + dx, j + dy
            if 0 <= x < _NX and 0 <= y < _NY:
                adj.append((j * _NX + i, y * _NX + x))
    return np.array(adj, dtype=np.int32)


_ADJ = _adj_np()
_NE = _ADJ.shape[0]

_SCAT = np.zeros((_NE, _NB * _NB), dtype=np.float32)
_SCAT[np.arange(_NE), _ADJ[:, 0] * _NB + _ADJ[:, 1]] = 1.0

_PERM = np.full((5 * 8 * 64,), -1, dtype=np.int32)
for _i in range(5):
    for _j in range(5):
        for _c in range(64):
            _PERM[(_i * 8 + _j) * 64 + _c] = _c * 25 + _i * 5 + _j


def _trunk_body(xp_ref, tm_ref, b12_ref, w3_ref, b3_ref, w4_ref, b4_ref,
                out_ref):
    x = xp_ref[0]
    xsh = jnp.concatenate([x[:, dx:dx + 128] for dx in range(9)],
                          axis=0).reshape(9, 128, 128)
    ps = []
    for t in range(15):
        xt = xsh[:, 8 * t:8 * t + 16, :].reshape(144, 128)
        yt = jnp.dot(tm_ref[...], xt, preferred_element_type=jnp.float32)
        q = jnp.max(yt.reshape(32, 8, 32, 4), axis=3)
        q = jnp.max(q.reshape(32, 2, 4, 32), axis=2)
        ps.append(q)
    pc = jnp.concatenate(ps, axis=1).reshape(32, 960)
    ii = jax.lax.broadcasted_iota(jnp.int32, (32, 32), 0)
    jj = jax.lax.broadcasted_iota(jnp.int32, (32, 32), 1)
    eye = jnp.where(ii == jj, 1.0, 0.0).astype(jnp.float32)
    x2 = jax.lax.dot_general(pc, eye, (((0,), (0,)), ((), ())),
                             preferred_element_type=jnp.float32)
    x2 = x2 + b12_ref[0][None, :]
    x2 = jnp.concatenate([x2, jnp.zeros((8, 32), jnp.float32)], axis=0)
    a3 = jnp.zeros((832, 64), jnp.float32) + b3_ref[0][None, :]
    for dy in range(5):
        xc = jnp.concatenate(
            [x2[dy * 32 + dx:dy * 32 + dx + 832, :] for dx in range(5)],
            axis=1)
        a3 = a3 + jnp.dot(xc, w3_ref[dy * 160:(dy + 1) * 160, :],
                          preferred_element_type=jnp.float32)
    a3 = jnp.concatenate([a3, jnp.zeros((8, 64), jnp.float32)], axis=0)
    a4 = jnp.zeros((704, 64), jnp.float32) + b4_ref[0][None, :]
    for dy in range(5):
        xc = jnp.concatenate(
            [a3[dy * 32 + dx:dy * 32 + dx + 704, :] for dx in range(5)],
            axis=1)
        a4 = a4 + jnp.dot(xc, w4_ref[dy * 320:(dy + 1) * 320, :],
                          preferred_element_type=jnp.float32)
    q = jnp.max(a4.reshape(22, 8, 4, 64), axis=2)
    q = q[0:20]
    q = jnp.max(q.reshape(5, 4, 8, 64), axis=1)
    out_ref[0] = q


def _heads_body(emb_ref, wc1_ref, bc1_ref, wc2_ref, bc2_ref,
                we1_ref, be1_ref, we2_ref, be2_ref, s_ref,
                a_ref, ie_ref):
    emb = emb_ref[...]
    hc = jnp.maximum(
        jnp.dot(emb, wc1_ref[...], preferred_element_type=jnp.float32)
        + bc1_ref[0][None, :], 0.0)
    ic = jax.nn.sigmoid(
        jnp.dot(hc, wc2_ref[...], preferred_element_type=jnp.float32)
        + bc2_ref[0][None, :])
    a_ref[...] = jnp.dot(ic, s_ref[...], preferred_element_type=jnp.float32)
    he = jnp.maximum(
        jnp.dot(emb, we1_ref[...], preferred_element_type=jnp.float32)
        + be1_ref[0][None, :], 0.0)
    ie_ref[...] = jax.nn.sigmoid(
        jnp.dot(he, we2_ref[...], preferred_element_type=jnp.float32)
        + be2_ref[0][None, :])


def _graph_body(a_ref, ier_ref, iec_ref, o_ref):
    r = a_ref[0]
    for _ in range(6):
        acc = r
        for k in range(_NB):
            acc = jnp.maximum(acc, r[:, k:k + 1] * r[k:k + 1, :])
        r = acc
    pair = r * ier_ref[0] * iec_ref[0]
    ii = jax.lax.broadcasted_iota(jnp.int32, (_NB, _NB), 0)
    jj = jax.lax.broadcasted_iota(jnp.int32, (_NB, _NB), 1)
    pair = jnp.where(ii != jj, pair, 0.0)
    o_ref[0] = jnp.max(pair, axis=(0, 1), keepdims=True)


def kernel(input, W1, b1, W2, b2, W3, b3, W4, b4, We1, be1, We2, be2,
           Wc1, bc1, Wc2, bc2):
    B = input.shape[0]
    f32 = jnp.float32

    w12 = jnp.zeros((32, 9, 9), f32)
    for u1 in range(5):
        for v1 in range(5):
            w12 = w12.at[:, u1:u1 + 5, v1:v1 + 5].add(
                jnp.einsum('ocuv,c->ouv', W2, W1[:, 0, u1, v1]))
    z = jnp.concatenate([w12.transpose(0, 2, 1),
                         jnp.zeros((32, 9, 8), f32)], axis=2)
    tm = jnp.tile(z, (1, 1, 8))[:, :, :128].reshape(32, 9, 8, 16)
    tm = tm.transpose(0, 2, 1, 3).reshape(256, 144)
    b12 = (b2 + W2.sum(axis=(2, 3)) @ b1).reshape(1, 32)
    w3m = W3.transpose(2, 3, 1, 0).reshape(800, 64)
    w4m = W4.transpose(2, 3, 1, 0).reshape(1600, 64)

    def _hwc(w):
        w = w.reshape(64, 5, 5, 256).transpose(1, 2, 0, 3)
        w = jnp.pad(w, ((0, 0), (0, 3), (0, 0), (0, 0)))
        return w.reshape(2560, 256)

    wc1h = _hwc(Wc1)
    we1h = _hwc(We1)

    xp = jnp.pad(input[:, 0], ((0, 0), (0, 0), (0, 8)))

    emb3 = pl.pallas_call(
        _trunk_body,
        grid=(B,),
        in_specs=[
            pl.BlockSpec((1, 128, 136), lambda b: (b, 0, 0)),
            pl.BlockSpec((256, 144), lambda b: (0, 0)),
            pl.BlockSpec((1, 32), lambda b: (0, 0)),
            pl.BlockSpec((800, 64), lambda b: (0, 0)),
            pl.BlockSpec((1, 64), lambda b: (0, 0)),
            pl.BlockSpec((1600, 64), lambda b: (0, 0)),
            pl.BlockSpec((1, 64), lambda b: (0, 0)),
        ],
        out_specs=pl.BlockSpec((1, 5, 8, 64), lambda b: (b, 0, 0, 0)),
        out_shape=jax.ShapeDtypeStruct((B, 5, 8, 64), f32),
    )(xp, tm, b12, w3m, b3.reshape(1, 64), w4m, b4.reshape(1, 64))

    emb = emb3.reshape(B, 2560)

    a_flat, ie = pl.pallas_call(
        _heads_body,
        out_shape=(jax.ShapeDtypeStruct((B, _NB * _NB), f32),
                   jax.ShapeDtypeStruct((B, _NB), f32)),
    )(emb, wc1h, bc1.reshape(1, 256), Wc2, bc2.reshape(1, _NE),
      we1h, be1.reshape(1, 256), We2, be2.reshape(1, _NB),
      jnp.asarray(_SCAT))

    a3 = a_flat.reshape(B, _NB, _NB)
    ier = ie.reshape(B, 1, _NB)
    iec = ie.reshape(B, _NB, 1)
    out = pl.pallas_call(
        _graph_body,
        grid=(B,),
        in_specs=[
            pl.BlockSpec((1, _NB, _NB), lambda b: (b, 0, 0)),
            pl.BlockSpec((1, 1, _NB), lambda b: (b, 0, 0)),
            pl.BlockSpec((1, _NB, 1), lambda b: (b, 0, 0)),
        ],
        out_specs=pl.BlockSpec((1, 1, 1), lambda b: (b, 0, 0)),
        out_shape=jax.ShapeDtypeStruct((B, 1, 1), f32),
    )(a3, ier, iec)
    return out.reshape(B)


# speedup vs baseline: 1.6997x; 1.0007x over previous
import itertools
from functools import partial

import numpy as np
import jax
import jax.numpy as jnp
from jax.experimental import pallas as pl

_NX, _NY = 6, 6
_NB = _NX * _NY


def _adj_np():
    adj = []
    for i, j in itertools.product(range(_NX), range(_NY)):
        for dx, dy in [(-1, 0), (0, -1), (0, 1), (1, 0)]:
            x, y = i + dx, j + dy
            if 0 <= x < _NX and 0 <= y < _NY:
                adj.append((j * _NX + i, y * _NX + x))
    return np.array(adj, dtype=np.int32)


_ADJ = _adj_np()
_NE = _ADJ.shape[0]

_SCAT = np.zeros((_NE, _NB * _NB), dtype=np.float32)
_SCAT[np.arange(_NE), _ADJ[:, 0] * _NB + _ADJ[:, 1]] = 1.0

_PERM = np.full((5 * 8 * 64,), -1, dtype=np.int32)
for _i in range(5):
    for _j in range(5):
        for _c in range(64):
            _PERM[(_i * 8 + _j) * 64 + _c] = _c * 25 + _i * 5 + _j


def _trunk_body(xp_ref, tm_ref, b12_ref, w3_ref, b3_ref, w4_ref, b4_ref,
                out_ref):
    x = xp_ref[0]
    xsh = jnp.concatenate([x[:, dx:dx + 128] for dx in range(9)],
                          axis=0).reshape(9, 128, 128).astype(jnp.bfloat16)
    ps = []
    for t in range(15):
        xt = xsh[:, 8 * t:8 * t + 16, :].reshape(144, 128)
        yt = jnp.dot(tm_ref[...], xt, preferred_element_type=jnp.float32)
        q = jnp.max(yt.reshape(32, 8, 32, 4), axis=3)
        q = jnp.max(q.reshape(32, 2, 4, 32), axis=2)
        ps.append(q)
    pc = jnp.concatenate(ps, axis=1).reshape(32, 960)
    ii = jax.lax.broadcasted_iota(jnp.int32, (32, 32), 0)
    jj = jax.lax.broadcasted_iota(jnp.int32, (32, 32), 1)
    eye = jnp.where(ii == jj, 1.0, 0.0).astype(jnp.float32)
    x2 = jax.lax.dot_general(pc, eye, (((0,), (0,)), ((), ())),
                             preferred_element_type=jnp.float32)
    x2 = (x2 + b12_ref[0][None, :]).astype(jnp.bfloat16)
    x2 = jnp.concatenate([x2, jnp.zeros((8, 32), jnp.bfloat16)], axis=0)
    a3 = jnp.zeros((832, 64), jnp.float32) + b3_ref[0][None, :]
    for dy in range(5):
        xc = jnp.concatenate(
            [x2[dy * 32 + dx:dy * 32 + dx + 832, :] for dx in range(5)],
            axis=1)
        a3 = a3 + jnp.dot(xc, w3_ref[dy * 160:(dy + 1) * 160, :],
                          preferred_element_type=jnp.float32)
    a3 = jnp.concatenate([a3.astype(jnp.bfloat16),
                          jnp.zeros((8, 64), jnp.bfloat16)], axis=0)
    a4 = jnp.zeros((704, 64), jnp.float32) + b4_ref[0][None, :]
    for dy in range(5):
        xc = jnp.concatenate(
            [a3[dy * 32 + dx:dy * 32 + dx + 704, :] for dx in range(5)],
            axis=1)
        a4 = a4 + jnp.dot(xc, w4_ref[dy * 320:(dy + 1) * 320, :],
                          preferred_element_type=jnp.float32)
    q = jnp.max(a4.reshape(22, 8, 4, 64), axis=2)
    q = q[0:20]
    q = jnp.max(q.reshape(5, 4, 8, 64), axis=1)
    out_ref[0] = q


def _heads_body(emb_ref, wc1_ref, bc1_ref, wc2_ref, bc2_ref,
                we1_ref, be1_ref, we2_ref, be2_ref, s_ref,
                a_ref, ie_ref):
    emb = emb_ref[...]
    hc = jnp.maximum(
        jnp.dot(emb, wc1_ref[...], preferred_element_type=jnp.float32)
        + bc1_ref[0][None, :], 0.0)
    ic = jax.nn.sigmoid(
        jnp.dot(hc, wc2_ref[...], preferred_element_type=jnp.float32)
        + bc2_ref[0][None, :])
    a_ref[...] = jnp.dot(ic, s_ref[...], preferred_element_type=jnp.float32)
    he = jnp.maximum(
        jnp.dot(emb, we1_ref[...], preferred_element_type=jnp.float32)
        + be1_ref[0][None, :], 0.0)
    ie_ref[...] = jax.nn.sigmoid(
        jnp.dot(he, we2_ref[...], preferred_element_type=jnp.float32)
        + be2_ref[0][None, :])


def _graph_body(a_ref, ier_ref, iec_ref, o_ref):
    r = a_ref[0]
    for _ in range(6):
        acc = r
        for k in range(_NB):
            acc = jnp.maximum(acc, r[:, k:k + 1] * r[k:k + 1, :])
        r = acc
    pair = r * ier_ref[0] * iec_ref[0]
    ii = jax.lax.broadcasted_iota(jnp.int32, (_NB, _NB), 0)
    jj = jax.lax.broadcasted_iota(jnp.int32, (_NB, _NB), 1)
    pair = jnp.where(ii != jj, pair, 0.0)
    o_ref[0] = jnp.max(pair, axis=(0, 1), keepdims=True)


def kernel(input, W1, b1, W2, b2, W3, b3, W4, b4, We1, be1, We2, be2,
           Wc1, bc1, Wc2, bc2):
    B = input.shape[0]
    f32 = jnp.float32

    w12 = jnp.zeros((32, 9, 9), f32)
    for u1 in range(5):
        for v1 in range(5):
            w12 = w12.at[:, u1:u1 + 5, v1:v1 + 5].add(
                jnp.einsum('ocuv,c->ouv', W2, W1[:, 0, u1, v1]))
    z = jnp.concatenate([w12.transpose(0, 2, 1),
                         jnp.zeros((32, 9, 8), f32)], axis=2)
    tm = jnp.tile(z, (1, 1, 8))[:, :, :128].reshape(32, 9, 8, 16)
    tm = tm.transpose(0, 2, 1, 3).reshape(256, 144).astype(jnp.bfloat16)
    b12 = (b2 + W2.sum(axis=(2, 3)) @ b1).reshape(1, 32)
    w3m = W3.transpose(2, 3, 1, 0).reshape(800, 64).astype(jnp.bfloat16)
    w4m = W4.transpose(2, 3, 1, 0).reshape(1600, 64).astype(jnp.bfloat16)

    def _hwc(w):
        w = w.reshape(64, 5, 5, 256).transpose(1, 2, 0, 3)
        w = jnp.pad(w, ((0, 0), (0, 3), (0, 0), (0, 0)))
        return w.reshape(2560, 256)

    wc1h = _hwc(Wc1)
    we1h = _hwc(We1)

    xp = jnp.pad(input[:, 0], ((0, 0), (0, 0), (0, 8)))

    emb3 = pl.pallas_call(
        _trunk_body,
        grid=(B,),
        in_specs=[
            pl.BlockSpec((1, 128, 136), lambda b: (b, 0, 0)),
            pl.BlockSpec((256, 144), lambda b: (0, 0)),
            pl.BlockSpec((1, 32), lambda b: (0, 0)),
            pl.BlockSpec((800, 64), lambda b: (0, 0)),
            pl.BlockSpec((1, 64), lambda b: (0, 0)),
            pl.BlockSpec((1600, 64), lambda b: (0, 0)),
            pl.BlockSpec((1, 64), lambda b: (0, 0)),
        ],
        out_specs=pl.BlockSpec((1, 5, 8, 64), lambda b: (b, 0, 0, 0)),
        out_shape=jax.ShapeDtypeStruct((B, 5, 8, 64), f32),
    )(xp, tm, b12, w3m, b3.reshape(1, 64), w4m, b4.reshape(1, 64))

    emb = emb3.reshape(B, 2560)

    a_flat, ie = pl.pallas_call(
        _heads_body,
        out_shape=(jax.ShapeDtypeStruct((B, _NB * _NB), f32),
                   jax.ShapeDtypeStruct((B, _NB), f32)),
    )(emb, wc1h, bc1.reshape(1, 256), Wc2, bc2.reshape(1, _NE),
      we1h, be1.reshape(1, 256), We2, be2.reshape(1, _NB),
      jnp.asarray(_SCAT))

    a3 = a_flat.reshape(B, _NB, _NB)
    ier = ie.reshape(B, 1, _NB)
    iec = ie.reshape(B, _NB, 1)
    out = pl.pallas_call(
        _graph_body,
        grid=(B,),
        in_specs=[
            pl.BlockSpec((1, _NB, _NB), lambda b: (b, 0, 0)),
            pl.BlockSpec((1, 1, _NB), lambda b: (b, 0, 0)),
            pl.BlockSpec((1, _NB, 1), lambda b: (b, 0, 0)),
        ],
        out_specs=pl.BlockSpec((1, 1, 1), lambda b: (b, 0, 0)),
        out_shape=jax.ShapeDtypeStruct((B, 1, 1), f32),
    )(a3, ier, iec)
    return out.reshape(B)


# speedup vs baseline: 3.6407x; 2.1420x over previous
import itertools
from functools import partial

import numpy as np
import jax
import jax.numpy as jnp
from jax.experimental import pallas as pl
from jax.experimental.pallas import tpu as pltpu

_NX, _NY = 6, 6
_NB = _NX * _NY


def _adj_np():
    adj = []
    for i, j in itertools.product(range(_NX), range(_NY)):
        for dx, dy in [(-1, 0), (0, -1), (0, 1), (1, 0)]:
            x, y = i + dx, j + dy
            if 0 <= x < _NX and 0 <= y < _NY:
                adj.append((j * _NX + i, y * _NX + x))
    return np.array(adj, dtype=np.int32)


_ADJ = _adj_np()
_NE = _ADJ.shape[0]

_SCAT = np.zeros((_NE, _NB * _NB), dtype=np.float32)
_SCAT[np.arange(_NE), _ADJ[:, 0] * _NB + _ADJ[:, 1]] = 1.0

_PERM = np.full((5 * 8 * 64,), -1, dtype=np.int32)
for _i in range(5):
    for _j in range(5):
        for _c in range(64):
            _PERM[(_i * 8 + _j) * 64 + _c] = _c * 25 + _i * 5 + _j


def _trunk_body(xp_ref, tm_ref, b12_ref, w3_ref, b3_ref, w4_ref, b4_ref,
                out_ref):
    x = xp_ref[0]
    xsh = jnp.concatenate([x[:, dx:dx + 128] for dx in range(9)],
                          axis=0).reshape(9, 128, 32, 4)
    xsh = jnp.transpose(xsh, (0, 1, 3, 2)).reshape(9, 128, 128)
    xsh = xsh.astype(jnp.bfloat16)
    ps = []
    for t in range(15):
        xt = xsh[:, 8 * t:8 * t + 16, :].reshape(144, 128)
        yt = jnp.dot(tm_ref[...], xt, preferred_element_type=jnp.float32)
        m = jnp.maximum(yt, pltpu.roll(yt, 64, 1))
        m = jnp.maximum(m, pltpu.roll(m, 32, 1))[:, 0:32]
        q = jnp.maximum(jnp.maximum(m[0:64], m[64:128]),
                        jnp.maximum(m[128:192], m[192:256]))
        ps.append(q.reshape(32, 2, 32))
    pc = jnp.concatenate(ps, axis=1).reshape(32, 960)
    ii = jax.lax.broadcasted_iota(jnp.int32, (32, 32), 0)
    jj = jax.lax.broadcasted_iota(jnp.int32, (32, 32), 1)
    eye = jnp.where(ii == jj, 1.0, 0.0).astype(jnp.float32)
    x2 = jax.lax.dot_general(pc, eye, (((0,), (0,)), ((), ())),
                             preferred_element_type=jnp.float32)
    x2 = (x2 + b12_ref[0][None, :]).astype(jnp.bfloat16)
    x2 = jnp.concatenate([x2, jnp.zeros((8, 32), jnp.bfloat16)], axis=0)
    a3 = jnp.zeros((832, 64), jnp.float32) + b3_ref[0][None, :]
    for dy in range(5):
        xc = jnp.concatenate(
            [x2[dy * 32 + dx:dy * 32 + dx + 832, :] for dx in range(5)],
            axis=1)
        a3 = a3 + jnp.dot(xc, w3_ref[dy * 160:(dy + 1) * 160, :],
                          preferred_element_type=jnp.float32)
    a3 = jnp.concatenate([a3.astype(jnp.bfloat16),
                          jnp.zeros((8, 64), jnp.bfloat16)], axis=0)
    a4 = jnp.zeros((704, 64), jnp.float32) + b4_ref[0][None, :]
    for dy in range(5):
        xc = jnp.concatenate(
            [a3[dy * 32 + dx:dy * 32 + dx + 704, :] for dx in range(5)],
            axis=1)
        a4 = a4 + jnp.dot(xc, w4_ref[dy * 320:(dy + 1) * 320, :],
                          preferred_element_type=jnp.float32)
    q = jnp.max(a4.reshape(22, 8, 4, 64), axis=2)
    q = q[0:20]
    q = jnp.max(q.reshape(5, 4, 8, 64), axis=1)
    out_ref[0] = q


def _heads_body(emb_ref, wc1_ref, bc1_ref, wc2_ref, bc2_ref,
                we1_ref, be1_ref, we2_ref, be2_ref, s_ref,
                a_ref, ie_ref):
    emb = emb_ref[...]
    hc = jnp.maximum(
        jnp.dot(emb, wc1_ref[...], preferred_element_type=jnp.float32)
        + bc1_ref[0][None, :], 0.0)
    ic = jax.nn.sigmoid(
        jnp.dot(hc, wc2_ref[...], preferred_element_type=jnp.float32)
        + bc2_ref[0][None, :])
    a_ref[...] = jnp.dot(ic, s_ref[...], preferred_element_type=jnp.float32)
    he = jnp.maximum(
        jnp.dot(emb, we1_ref[...], preferred_element_type=jnp.float32)
        + be1_ref[0][None, :], 0.0)
    ie_ref[...] = jax.nn.sigmoid(
        jnp.dot(he, we2_ref[...], preferred_element_type=jnp.float32)
        + be2_ref[0][None, :])


def _graph_body(a_ref, ier_ref, iec_ref, o_ref):
    r = a_ref[0]
    for _ in range(6):
        acc = r
        for k in range(_NB):
            acc = jnp.maximum(acc, r[:, k:k + 1] * r[k:k + 1, :])
        r = acc
    pair = r * ier_ref[0] * iec_ref[0]
    ii = jax.lax.broadcasted_iota(jnp.int32, (_NB, _NB), 0)
    jj = jax.lax.broadcasted_iota(jnp.int32, (_NB, _NB), 1)
    pair = jnp.where(ii != jj, pair, 0.0)
    o_ref[0] = jnp.max(pair, axis=(0, 1), keepdims=True)


def kernel(input, W1, b1, W2, b2, W3, b3, W4, b4, We1, be1, We2, be2,
           Wc1, bc1, Wc2, bc2):
    B = input.shape[0]
    f32 = jnp.float32

    w12 = jnp.zeros((32, 9, 9), f32)
    for u1 in range(5):
        for v1 in range(5):
            w12 = w12.at[:, u1:u1 + 5, v1:v1 + 5].add(
                jnp.einsum('ocuv,c->ouv', W2, W1[:, 0, u1, v1]))
    z = jnp.concatenate([w12.transpose(0, 2, 1),
                         jnp.zeros((32, 9, 8), f32)], axis=2)
    tm = jnp.tile(z, (1, 1, 8))[:, :, :128].reshape(32, 9, 8, 16)
    tm = tm.reshape(32, 9, 2, 4, 16).transpose(3, 0, 2, 1, 4)
    tm = tm.reshape(256, 144).astype(jnp.bfloat16)
    b12 = (b2 + W2.sum(axis=(2, 3)) @ b1).reshape(1, 32)
    w3m = W3.transpose(2, 3, 1, 0).reshape(800, 64).astype(jnp.bfloat16)
    w4m = W4.transpose(2, 3, 1, 0).reshape(1600, 64).astype(jnp.bfloat16)

    def _hwc(w):
        w = w.reshape(64, 5, 5, 256).transpose(1, 2, 0, 3)
        w = jnp.pad(w, ((0, 0), (0, 3), (0, 0), (0, 0)))
        return w.reshape(2560, 256)

    wc1h = _hwc(Wc1)
    we1h = _hwc(We1)

    xp = jnp.pad(input[:, 0], ((0, 0), (0, 0), (0, 8)))

    emb3 = pl.pallas_call(
        _trunk_body,
        grid=(B,),
        in_specs=[
            pl.BlockSpec((1, 128, 136), lambda b: (b, 0, 0)),
            pl.BlockSpec((256, 144), lambda b: (0, 0)),
            pl.BlockSpec((1, 32), lambda b: (0, 0)),
            pl.BlockSpec((800, 64), lambda b: (0, 0)),
            pl.BlockSpec((1, 64), lambda b: (0, 0)),
            pl.BlockSpec((1600, 64), lambda b: (0, 0)),
            pl.BlockSpec((1, 64), lambda b: (0, 0)),
        ],
        out_specs=pl.BlockSpec((1, 5, 8, 64), lambda b: (b, 0, 0, 0)),
        out_shape=jax.ShapeDtypeStruct((B, 5, 8, 64), f32),
    )(xp, tm, b12, w3m, b3.reshape(1, 64), w4m, b4.reshape(1, 64))

    emb = emb3.reshape(B, 2560)

    a_flat, ie = pl.pallas_call(
        _heads_body,
        out_shape=(jax.ShapeDtypeStruct((B, _NB * _NB), f32),
                   jax.ShapeDtypeStruct((B, _NB), f32)),
    )(emb, wc1h, bc1.reshape(1, 256), Wc2, bc2.reshape(1, _NE),
      we1h, be1.reshape(1, 256), We2, be2.reshape(1, _NB),
      jnp.asarray(_SCAT))

    a3 = a_flat.reshape(B, _NB, _NB)
    ier = ie.reshape(B, 1, _NB)
    iec = ie.reshape(B, _NB, 1)
    out = pl.pallas_call(
        _graph_body,
        grid=(B,),
        in_specs=[
            pl.BlockSpec((1, _NB, _NB), lambda b: (b, 0, 0)),
            pl.BlockSpec((1, 1, _NB), lambda b: (b, 0, 0)),
            pl.BlockSpec((1, _NB, 1), lambda b: (b, 0, 0)),
        ],
        out_specs=pl.BlockSpec((1, 1, 1), lambda b: (b, 0, 0)),
        out_shape=jax.ShapeDtypeStruct((B, 1, 1), f32),
    )(a3, ier, iec)
    return out.reshape(B)


# speedup vs baseline: 3.9077x; 1.0733x over previous
import itertools
from functools import partial

import numpy as np
import jax
import jax.numpy as jnp
from jax.experimental import pallas as pl
from jax.experimental.pallas import tpu as pltpu
from jax.experimental.pallas import tpu_sc as plsc

_NX, _NY = 6, 6
_NB = _NX * _NY


def _adj_np():
    adj = []
    for i, j in itertools.product(range(_NX), range(_NY)):
        for dx, dy in [(-1, 0), (0, -1), (0, 1), (1, 0)]:
            x, y = i + dx, j + dy
            if 0 <= x < _NX and 0 <= y < _NY:
                adj.append((j * _NX + i, y * _NX + x))
    return np.array(adj, dtype=np.int32)


_ADJ = _adj_np()
_NE = _ADJ.shape[0]

_SCAT = np.zeros((_NE, _NB * _NB), dtype=np.float32)
_SCAT[np.arange(_NE), _ADJ[:, 0] * _NB + _ADJ[:, 1]] = 1.0

_PERM = np.full((5 * 8 * 64,), -1, dtype=np.int32)
for _i in range(5):
    for _j in range(5):
        for _c in range(64):
            _PERM[(_i * 8 + _j) * 64 + _c] = _c * 25 + _i * 5 + _j


def _trunk_body(xp_ref, tm_ref, b12_ref, w3_ref, b3_ref, w4_ref, b4_ref,
                out_ref):
    x = xp_ref[0]
    xsh = jnp.concatenate([x[:, dx:dx + 128] for dx in range(9)],
                          axis=0).reshape(9, 128, 32, 4)
    xsh = jnp.transpose(xsh, (0, 1, 3, 2)).reshape(9, 128, 128)
    xsh = xsh.astype(jnp.bfloat16)
    ps = []
    for t in range(15):
        xt = xsh[:, 8 * t:8 * t + 16, :].reshape(144, 128)
        yt = jnp.dot(tm_ref[...], xt, preferred_element_type=jnp.float32)
        m = jnp.maximum(yt, pltpu.roll(yt, 64, 1))
        m = jnp.maximum(m, pltpu.roll(m, 32, 1))[:, 0:32]
        q = jnp.maximum(jnp.maximum(m[0:64], m[64:128]),
                        jnp.maximum(m[128:192], m[192:256]))
        ps.append(q.reshape(32, 2, 32))
    pc = jnp.concatenate(ps, axis=1).reshape(32, 960)
    ii = jax.lax.broadcasted_iota(jnp.int32, (32, 32), 0)
    jj = jax.lax.broadcasted_iota(jnp.int32, (32, 32), 1)
    eye = jnp.where(ii == jj, 1.0, 0.0).astype(jnp.float32)
    x2 = jax.lax.dot_general(pc, eye, (((0,), (0,)), ((), ())),
                             preferred_element_type=jnp.float32)
    x2 = (x2 + b12_ref[0][None, :]).astype(jnp.bfloat16)
    x2 = jnp.concatenate([x2, jnp.zeros((8, 32), jnp.bfloat16)], axis=0)
    a3 = jnp.zeros((832, 64), jnp.float32) + b3_ref[0][None, :]
    for dy in range(5):
        xc = jnp.concatenate(
            [x2[dy * 32 + dx:dy * 32 + dx + 832, :] for dx in range(5)],
            axis=1)
        a3 = a3 + jnp.dot(xc, w3_ref[dy * 160:(dy + 1) * 160, :],
                          preferred_element_type=jnp.float32)
    a3 = jnp.concatenate([a3.astype(jnp.bfloat16),
                          jnp.zeros((8, 64), jnp.bfloat16)], axis=0)
    a4 = jnp.zeros((704, 64), jnp.float32) + b4_ref[0][None, :]
    for dy in range(5):
        xc = jnp.concatenate(
            [a3[dy * 32 + dx:dy * 32 + dx + 704, :] for dx in range(5)],
            axis=1)
        a4 = a4 + jnp.dot(xc, w4_ref[dy * 320:(dy + 1) * 320, :],
                          preferred_element_type=jnp.float32)
    q = jnp.max(a4.reshape(22, 8, 4, 64), axis=2)
    q = q[0:20]
    q = jnp.max(q.reshape(5, 4, 8, 64), axis=1)
    out_ref[0] = q


def _heads_body(emb_ref, wc1_ref, bc1_ref, wc2_ref, bc2_ref,
                we1_ref, be1_ref, we2_ref, be2_ref, s_ref,
                a_ref, ie_ref):
    emb = emb_ref[...]
    hc = jnp.maximum(
        jnp.dot(emb, wc1_ref[...], preferred_element_type=jnp.float32)
        + bc1_ref[0][None, :], 0.0)
    ic = jax.nn.sigmoid(
        jnp.dot(hc, wc2_ref[...], preferred_element_type=jnp.float32)
        + bc2_ref[0][None, :])
    a_ref[...] = jnp.dot(ic, s_ref[...], preferred_element_type=jnp.float32)
    he = jnp.maximum(
        jnp.dot(emb, we1_ref[...], preferred_element_type=jnp.float32)
        + be1_ref[0][None, :], 0.0)
    ie_ref[...] = jax.nn.sigmoid(
        jnp.dot(he, we2_ref[...], preferred_element_type=jnp.float32)
        + be2_ref[0][None, :])


def _graph_sc_body(a_hbm, ie_hbm, out_hbm, ra, rb, iev, outv, redv):
    wid = jax.lax.axis_index("s") * 2 + jax.lax.axis_index("c")
    pltpu.sync_copy(a_hbm.at[wid], ra)
    pltpu.sync_copy(ie_hbm.at[wid], iev)
    bufs = [ra, rb]
    for rnd in range(6):
        src, dst = bufs[rnd % 2], bufs[(rnd + 1) % 2]

        def row_update(i, _, src=src, dst=dst):
            row = [src[i, 16 * c:16 * (c + 1)] for c in range(3)]
            accs = list(row)
            for k in range(36):
                rik = row[k // 16][k % 16]
                for c in range(3):
                    accs[c] = jnp.maximum(accs[c],
                                          rik * src[k, 16 * c:16 * (c + 1)])
            for c in range(3):
                dst[i, 16 * c:16 * (c + 1)] = accs[c]
            return 0

        jax.lax.fori_loop(0, 36, row_update, 0)
    def pair_update(i, best):
        ei = iev[pl.ds(i, 16)][0]
        for c in range(3):
            jidx = jax.lax.iota(jnp.int32, 16) + 16 * c
            v = ra[i, 16 * c:16 * (c + 1)] * iev[16 * c:16 * (c + 1)] * ei
            best = jnp.maximum(best, jnp.where(jidx == i, 0.0, v))
        return best

    best = jax.lax.fori_loop(0, 36, pair_update, jnp.zeros((16,), jnp.float32))
    redv[16:32] = jnp.zeros((16,), jnp.float32)
    v = best
    for sh in (8, 4, 2, 1):
        redv[0:16] = v
        v = jnp.maximum(v, redv[pl.ds(sh, 16)])
    outv[...] = jnp.broadcast_to(v[0], (16,))
    pltpu.sync_copy(outv, out_hbm.at[wid])


def _graph_body(a_ref, ier_ref, iec_ref, o_ref):
    r = a_ref[0]
    for _ in range(6):
        acc = r
        for k in range(_NB):
            acc = jnp.maximum(acc, r[:, k:k + 1] * r[k:k + 1, :])
        r = acc
    pair = r * ier_ref[0] * iec_ref[0]
    ii = jax.lax.broadcasted_iota(jnp.int32, (_NB, _NB), 0)
    jj = jax.lax.broadcasted_iota(jnp.int32, (_NB, _NB), 1)
    pair = jnp.where(ii != jj, pair, 0.0)
    o_ref[0] = jnp.max(pair, axis=(0, 1), keepdims=True)


def kernel(input, W1, b1, W2, b2, W3, b3, W4, b4, We1, be1, We2, be2,
           Wc1, bc1, Wc2, bc2):
    B = input.shape[0]
    f32 = jnp.float32

    w12 = jnp.zeros((32, 9, 9), f32)
    for u1 in range(5):
        for v1 in range(5):
            w12 = w12.at[:, u1:u1 + 5, v1:v1 + 5].add(
                jnp.einsum('ocuv,c->ouv', W2, W1[:, 0, u1, v1]))
    z = jnp.concatenate([w12.transpose(0, 2, 1),
                         jnp.zeros((32, 9, 8), f32)], axis=2)
    tm = jnp.tile(z, (1, 1, 8))[:, :, :128].reshape(32, 9, 8, 16)
    tm = tm.reshape(32, 9, 2, 4, 16).transpose(3, 0, 2, 1, 4)
    tm = tm.reshape(256, 144).astype(jnp.bfloat16)
    b12 = (b2 + W2.sum(axis=(2, 3)) @ b1).reshape(1, 32)
    w3m = W3.transpose(2, 3, 1, 0).reshape(800, 64).astype(jnp.bfloat16)
    w4m = W4.transpose(2, 3, 1, 0).reshape(1600, 64).astype(jnp.bfloat16)

    def _hwc(w):
        w = w.reshape(64, 5, 5, 256).transpose(1, 2, 0, 3)
        w = jnp.pad(w, ((0, 0), (0, 3), (0, 0), (0, 0)))
        return w.reshape(2560, 256)

    wc1h = _hwc(Wc1)
    we1h = _hwc(We1)

    xp = jnp.pad(input[:, 0], ((0, 0), (0, 0), (0, 8)))

    emb3 = pl.pallas_call(
        _trunk_body,
        grid=(B,),
        in_specs=[
            pl.BlockSpec((1, 128, 136), lambda b: (b, 0, 0)),
            pl.BlockSpec((256, 144), lambda b: (0, 0)),
            pl.BlockSpec((1, 32), lambda b: (0, 0)),
            pl.BlockSpec((800, 64), lambda b: (0, 0)),
            pl.BlockSpec((1, 64), lambda b: (0, 0)),
            pl.BlockSpec((1600, 64), lambda b: (0, 0)),
            pl.BlockSpec((1, 64), lambda b: (0, 0)),
        ],
        out_specs=pl.BlockSpec((1, 5, 8, 64), lambda b: (b, 0, 0, 0)),
        out_shape=jax.ShapeDtypeStruct((B, 5, 8, 64), f32),
    )(xp, tm, b12, w3m, b3.reshape(1, 64), w4m, b4.reshape(1, 64))

    emb = emb3.reshape(B, 2560)

    a_flat, ie = pl.pallas_call(
        _heads_body,
        out_shape=(jax.ShapeDtypeStruct((B, _NB * _NB), f32),
                   jax.ShapeDtypeStruct((B, _NB), f32)),
    )(emb, wc1h, bc1.reshape(1, 256), Wc2, bc2.reshape(1, _NE),
      we1h, be1.reshape(1, 256), We2, be2.reshape(1, _NB),
      jnp.asarray(_SCAT))

    a_pad = jnp.pad(a_flat.reshape(B, _NB, _NB), ((0, 0), (0, 0), (0, 12)))
    ie_pad = jnp.pad(ie, ((0, 0), (0, 20)))
    out16 = pl.kernel(
        _graph_sc_body,
        out_type=jax.ShapeDtypeStruct((B, 16), f32),
        mesh=plsc.VectorSubcoreMesh(core_axis_name="c", subcore_axis_name="s"),
        scratch_types=[
            pltpu.VMEM((_NB, 48), f32),
            pltpu.VMEM((_NB, 48), f32),
            pltpu.VMEM((56,), f32),
            pltpu.VMEM((16,), f32),
            pltpu.VMEM((32,), f32),
        ],
    )(a_pad, ie_pad)
    return out16[:, 0]


# speedup vs baseline: 3.9159x; 1.0021x over previous
import itertools
from functools import partial

import numpy as np
import jax
import jax.numpy as jnp
from jax.experimental import pallas as pl
from jax.experimental.pallas import tpu as pltpu
from jax.experimental.pallas import tpu_sc as plsc

_NX, _NY = 6, 6
_NB = _NX * _NY


def _adj_np():
    adj = []
    for i, j in itertools.product(range(_NX), range(_NY)):
        for dx, dy in [(-1, 0), (0, -1), (0, 1), (1, 0)]:
            x, y = i + dx, j + dy
            if 0 <= x < _NX and 0 <= y < _NY:
                adj.append((j * _NX + i, y * _NX + x))
    return np.array(adj, dtype=np.int32)


_ADJ = _adj_np()
_NE = _ADJ.shape[0]

_SCAT = np.zeros((_NE, _NB * _NB), dtype=np.float32)
_SCAT[np.arange(_NE), _ADJ[:, 0] * _NB + _ADJ[:, 1]] = 1.0

_PERM = np.full((5 * 8 * 64,), -1, dtype=np.int32)
for _i in range(5):
    for _j in range(5):
        for _c in range(64):
            _PERM[(_i * 8 + _j) * 64 + _c] = _c * 25 + _i * 5 + _j


def _trunk_body(xp_ref, tm_ref, b12_ref, w3_ref, b3_ref, w4_ref, b4_ref,
                out_ref):
    x = xp_ref[0]
    xsh = jnp.concatenate([x[:, dx:dx + 128] for dx in range(9)],
                          axis=0).reshape(9, 128, 32, 4)
    xsh = jnp.transpose(xsh, (0, 1, 3, 2)).reshape(9, 128, 128)
    xsh = xsh.astype(jnp.bfloat16)
    ps = []
    for t in range(15):
        xt = xsh[:, 8 * t:8 * t + 16, :].reshape(144, 128)
        yt = jnp.dot(tm_ref[...], xt, preferred_element_type=jnp.float32)
        m = jnp.maximum(yt, pltpu.roll(yt, 64, 1))
        m = jnp.maximum(m, pltpu.roll(m, 32, 1))[:, 0:32]
        q = jnp.maximum(jnp.maximum(m[0:64], m[64:128]),
                        jnp.maximum(m[128:192], m[192:256]))
        ps.append(q.reshape(32, 2, 32))
    pc = jnp.concatenate(ps, axis=1).reshape(32, 960)
    ii = jax.lax.broadcasted_iota(jnp.int32, (32, 32), 0)
    jj = jax.lax.broadcasted_iota(jnp.int32, (32, 32), 1)
    eye = jnp.where(ii == jj, 1.0, 0.0).astype(jnp.float32)
    x2 = jax.lax.dot_general(pc, eye, (((0,), (0,)), ((), ())),
                             preferred_element_type=jnp.float32)
    x2 = (x2 + b12_ref[0][None, :]).astype(jnp.bfloat16)
    x2 = jnp.concatenate([x2, jnp.zeros((8, 32), jnp.bfloat16)], axis=0)
    x2sh = jnp.concatenate([x2[dx:dx + 960, :] for dx in range(5)],
                           axis=0).reshape(5, 960, 32)
    a3 = jnp.zeros((832, 64), jnp.float32) + b3_ref[0][None, :]
    for dy in range(5):
        xc = jnp.concatenate(
            [x2sh[dx, dy * 32:dy * 32 + 832, :] for dx in range(5)],
            axis=1)
        a3 = a3 + jnp.dot(xc, w3_ref[dy * 160:(dy + 1) * 160, :],
                          preferred_element_type=jnp.float32)
    a3 = jnp.concatenate([a3.astype(jnp.bfloat16),
                          jnp.zeros((8, 64), jnp.bfloat16)], axis=0)
    a3sh = jnp.concatenate([a3[dx:dx + 832, :] for dx in range(5)],
                           axis=0).reshape(5, 832, 64)
    a4 = jnp.zeros((704, 64), jnp.float32) + b4_ref[0][None, :]
    for dy in range(5):
        xc = jnp.concatenate(
            [a3sh[dx, dy * 32:dy * 32 + 704, :] for dx in range(5)],
            axis=1)
        a4 = a4 + jnp.dot(xc, w4_ref[dy * 320:(dy + 1) * 320, :],
                          preferred_element_type=jnp.float32)
    q = jnp.max(a4.reshape(22, 8, 4, 64), axis=2)
    q = q[0:20]
    q = jnp.max(q.reshape(5, 4, 8, 64), axis=1)
    out_ref[0] = q


def _heads_body(emb_ref, wc1_ref, bc1_ref, wc2_ref, bc2_ref,
                we1_ref, be1_ref, we2_ref, be2_ref, s_ref,
                a_ref, ie_ref):
    emb = emb_ref[...]
    hc = jnp.maximum(
        jnp.dot(emb, wc1_ref[...], preferred_element_type=jnp.float32)
        + bc1_ref[0][None, :], 0.0)
    ic = jax.nn.sigmoid(
        jnp.dot(hc, wc2_ref[...], preferred_element_type=jnp.float32)
        + bc2_ref[0][None, :])
    a_ref[...] = jnp.dot(ic, s_ref[...], preferred_element_type=jnp.float32)
    he = jnp.maximum(
        jnp.dot(emb, we1_ref[...], preferred_element_type=jnp.float32)
        + be1_ref[0][None, :], 0.0)
    ie_ref[...] = jax.nn.sigmoid(
        jnp.dot(he, we2_ref[...], preferred_element_type=jnp.float32)
        + be2_ref[0][None, :])


def _graph_sc_body(a_hbm, ie_hbm, out_hbm, ra, rb, iev, outv, redv):
    wid = jax.lax.axis_index("s") * 2 + jax.lax.axis_index("c")
    pltpu.sync_copy(a_hbm.at[wid], ra)
    pltpu.sync_copy(ie_hbm.at[wid], iev)
    bufs = [ra, rb]
    for rnd in range(6):
        src, dst = bufs[rnd % 2], bufs[(rnd + 1) % 2]

        def row_update(i, _, src=src, dst=dst):
            row = [src[i, 16 * c:16 * (c + 1)] for c in range(3)]
            accs = list(row)
            for k in range(36):
                rik = row[k // 16][k % 16]
                for c in range(3):
                    accs[c] = jnp.maximum(accs[c],
                                          rik * src[k, 16 * c:16 * (c + 1)])
            for c in range(3):
                dst[i, 16 * c:16 * (c + 1)] = accs[c]
            return 0

        jax.lax.fori_loop(0, 36, row_update, 0)
    def pair_update(i, best):
        ei = iev[pl.ds(i, 16)][0]
        for c in range(3):
            jidx = jax.lax.iota(jnp.int32, 16) + 16 * c
            v = ra[i, 16 * c:16 * (c + 1)] * iev[16 * c:16 * (c + 1)] * ei
            best = jnp.maximum(best, jnp.where(jidx == i, 0.0, v))
        return best

    best = jax.lax.fori_loop(0, 36, pair_update, jnp.zeros((16,), jnp.float32))
    redv[16:32] = jnp.zeros((16,), jnp.float32)
    v = best
    for sh in (8, 4, 2, 1):
        redv[0:16] = v
        v = jnp.maximum(v, redv[pl.ds(sh, 16)])
    outv[...] = jnp.broadcast_to(v[0], (16,))
    pltpu.sync_copy(outv, out_hbm.at[wid])


def _graph_body(a_ref, ier_ref, iec_ref, o_ref):
    r = a_ref[0]
    for _ in range(6):
        acc = r
        for k in range(_NB):
            acc = jnp.maximum(acc, r[:, k:k + 1] * r[k:k + 1, :])
        r = acc
    pair = r * ier_ref[0] * iec_ref[0]
    ii = jax.lax.broadcasted_iota(jnp.int32, (_NB, _NB), 0)
    jj = jax.lax.broadcasted_iota(jnp.int32, (_NB, _NB), 1)
    pair = jnp.where(ii != jj, pair, 0.0)
    o_ref[0] = jnp.max(pair, axis=(0, 1), keepdims=True)


def kernel(input, W1, b1, W2, b2, W3, b3, W4, b4, We1, be1, We2, be2,
           Wc1, bc1, Wc2, bc2):
    B = input.shape[0]
    f32 = jnp.float32

    w12 = jnp.zeros((32, 9, 9), f32)
    for u1 in range(5):
        for v1 in range(5):
            w12 = w12.at[:, u1:u1 + 5, v1:v1 + 5].add(
                jnp.einsum('ocuv,c->ouv', W2, W1[:, 0, u1, v1]))
    z = jnp.concatenate([w12.transpose(0, 2, 1),
                         jnp.zeros((32, 9, 8), f32)], axis=2)
    tm = jnp.tile(z, (1, 1, 8))[:, :, :128].reshape(32, 9, 8, 16)
    tm = tm.reshape(32, 9, 2, 4, 16).transpose(3, 0, 2, 1, 4)
    tm = tm.reshape(256, 144).astype(jnp.bfloat16)
    b12 = (b2 + W2.sum(axis=(2, 3)) @ b1).reshape(1, 32)
    w3m = W3.transpose(2, 3, 1, 0).reshape(800, 64).astype(jnp.bfloat16)
    w4m = W4.transpose(2, 3, 1, 0).reshape(1600, 64).astype(jnp.bfloat16)

    def _hwc(w):
        w = w.reshape(64, 5, 5, 256).transpose(1, 2, 0, 3)
        w = jnp.pad(w, ((0, 0), (0, 3), (0, 0), (0, 0)))
        return w.reshape(2560, 256)

    wc1h = _hwc(Wc1)
    we1h = _hwc(We1)

    xp = jnp.pad(input[:, 0], ((0, 0), (0, 0), (0, 8)))

    emb3 = pl.pallas_call(
        _trunk_body,
        grid=(B,),
        in_specs=[
            pl.BlockSpec((1, 128, 136), lambda b: (b, 0, 0)),
            pl.BlockSpec((256, 144), lambda b: (0, 0)),
            pl.BlockSpec((1, 32), lambda b: (0, 0)),
            pl.BlockSpec((800, 64), lambda b: (0, 0)),
            pl.BlockSpec((1, 64), lambda b: (0, 0)),
            pl.BlockSpec((1600, 64), lambda b: (0, 0)),
            pl.BlockSpec((1, 64), lambda b: (0, 0)),
        ],
        out_specs=pl.BlockSpec((1, 5, 8, 64), lambda b: (b, 0, 0, 0)),
        out_shape=jax.ShapeDtypeStruct((B, 5, 8, 64), f32),
    )(xp, tm, b12, w3m, b3.reshape(1, 64), w4m, b4.reshape(1, 64))

    emb = emb3.reshape(B, 2560)

    a_flat, ie = pl.pallas_call(
        _heads_body,
        out_shape=(jax.ShapeDtypeStruct((B, _NB * _NB), f32),
                   jax.ShapeDtypeStruct((B, _NB), f32)),
    )(emb, wc1h, bc1.reshape(1, 256), Wc2, bc2.reshape(1, _NE),
      we1h, be1.reshape(1, 256), We2, be2.reshape(1, _NB),
      jnp.asarray(_SCAT))

    a_pad = jnp.pad(a_flat.reshape(B, _NB, _NB), ((0, 0), (0, 0), (0, 12)))
    ie_pad = jnp.pad(ie, ((0, 0), (0, 20)))
    out16 = pl.kernel(
        _graph_sc_body,
        out_type=jax.ShapeDtypeStruct((B, 16), f32),
        mesh=plsc.VectorSubcoreMesh(core_axis_name="c", subcore_axis_name="s"),
        scratch_types=[
            pltpu.VMEM((_NB, 48), f32),
            pltpu.VMEM((_NB, 48), f32),
            pltpu.VMEM((56,), f32),
            pltpu.VMEM((16,), f32),
            pltpu.VMEM((32,), f32),
        ],
    )(a_pad, ie_pad)
    return out16[:, 0]
